# baseline (speedup 1.0000x reference)
import jax
import jax.numpy as jnp
from jax.experimental import pallas as pl
from jax.experimental.pallas import tpu as pltpu

CHUNKS = (8000, 22000, 22000, 22000, 22000, 4000)
CH_MAX = max(CHUNKS)
OFFS = []
_o = 0
for _c in CHUNKS:
    OFFS.append(_o)
    _o += _c
OFFS = tuple(OFFS)


def _body(x_hbm, w_ref, b_ref, o_hbm, x_buf, o_buf, in_sems, out_sems):
    n_ch = len(CHUNKS)

    def start_load(j):
        cp = pltpu.make_async_copy(
            x_hbm.at[pl.ds(OFFS[j], CHUNKS[j]), :],
            x_buf.at[j % 2, pl.ds(0, CHUNKS[j]), :],
            in_sems.at[j % 2],
        )
        cp.start()
        return cp

    in_cps = [start_load(i) for i in range(min(2, n_ch))]
    out_cps = [None] * n_ch
    for i in range(n_ch):
        slot = i % 2
        in_cps[i].wait()
        if i >= 2:
            out_cps[i - 2].wait()
        xs = x_buf[slot, pl.ds(0, CHUNKS[i]), :]
        res = (
            jax.lax.dot_general(
                xs,
                w_ref[...],
                (((1,), (1,)), ((), ())),
                preferred_element_type=jnp.float32,
            )
            + b_ref[...]
        )
        o_buf[slot, pl.ds(0, CHUNKS[i]), :] = res
        ocp = pltpu.make_async_copy(
            o_buf.at[slot, pl.ds(0, CHUNKS[i]), :],
            o_hbm.at[pl.ds(OFFS[i], CHUNKS[i]), :],
            out_sems.at[slot],
        )
        ocp.start()
        out_cps[i] = ocp
        if i + 2 < n_ch:
            in_cps.append(start_load(i + 2))
    for i in range(max(0, n_ch - 2), n_ch):
        out_cps[i].wait()


def kernel(x, W, b):
    n, hidden = x.shape
    out_dim = W.shape[0]
    b2 = b.reshape(1, out_dim)
    return pl.pallas_call(
        _body,
        in_specs=[
            pl.BlockSpec(memory_space=pl.MemorySpace.ANY),
            pl.BlockSpec(memory_space=pltpu.MemorySpace.VMEM),
            pl.BlockSpec(memory_space=pltpu.MemorySpace.VMEM),
        ],
        out_specs=pl.BlockSpec(memory_space=pl.MemorySpace.ANY),
        out_shape=jax.ShapeDtypeStruct((n, out_dim), jnp.float32),
        scratch_shapes=[
            pltpu.VMEM((2, CH_MAX, hidden), jnp.float32),
            pltpu.VMEM((2, CH_MAX, out_dim), jnp.float32),
            pltpu.SemaphoreType.DMA((2,)),
            pltpu.SemaphoreType.DMA((2,)),
        ],
    )(x, W, b2)


# speedup vs baseline: 1.0304x; 1.0304x over previous
import jax
import jax.numpy as jnp
from jax.experimental import pallas as pl
from jax.experimental.pallas import tpu as pltpu

CHUNKS = (8000, 16000, 16000, 16000, 16000, 16000, 12000)
CH_MAX = max(CHUNKS)
OFFS = []
_o = 0
for _c in CHUNKS:
    OFFS.append(_o)
    _o += _c
OFFS = tuple(OFFS)
N_CH = len(CHUNKS)


def _body(x_hbm, w_ref, b_ref, o_hbm, x_buf, o_buf, in_sems, out_sems):
    def start_load(j):
        s = j % 3
        rows = CHUNKS[j]
        h = rows // 2
        cps = []
        for k, (r0, rn) in enumerate(((0, h), (h, rows - h))):
            cp = pltpu.make_async_copy(
                x_hbm.at[pl.ds(OFFS[j] + r0, rn), :],
                x_buf.at[s, pl.ds(r0, rn), :],
                in_sems.at[s, k],
            )
            cp.start()
            cps.append(cp)
        return cps

    def start_store(j):
        s = j % 2
        rows = CHUNKS[j]
        h = rows // 2
        cps = []
        for k, (r0, rn) in enumerate(((0, h), (h, rows - h))):
            cp = pltpu.make_async_copy(
                o_buf.at[s, pl.ds(r0, rn), :],
                o_hbm.at[pl.ds(OFFS[j] + r0, rn), :],
                out_sems.at[s, k],
            )
            cp.start()
            cps.append(cp)
        return cps

    in_cps = [start_load(j) for j in range(min(3, N_CH))]
    out_cps = [None] * N_CH
    for i in range(N_CH):
        for cp in in_cps[i]:
            cp.wait()
        if i >= 2:
            for cp in out_cps[i - 2]:
                cp.wait()
        xs = x_buf[i % 3, pl.ds(0, CHUNKS[i]), :]
        res = (
            jax.lax.dot_general(
                xs,
                w_ref[...],
                (((1,), (1,)), ((), ())),
                preferred_element_type=jnp.float32,
            )
            + b_ref[...]
        )
        o_buf[i % 2, pl.ds(0, CHUNKS[i]), :] = res
        out_cps[i] = start_store(i)
        if i + 3 < N_CH:
            in_cps.append(start_load(i + 3))
    for i in range(max(0, N_CH - 2), N_CH):
        for cp in out_cps[i]:
            cp.wait()


def kernel(x, W, b):
    n, hidden = x.shape
    out_dim = W.shape[0]
    b2 = b.reshape(1, out_dim)
    return pl.pallas_call(
        _body,
        in_specs=[
            pl.BlockSpec(memory_space=pl.MemorySpace.ANY),
            pl.BlockSpec(memory_space=pltpu.MemorySpace.VMEM),
            pl.BlockSpec(memory_space=pltpu.MemorySpace.VMEM),
        ],
        out_specs=pl.BlockSpec(memory_space=pl.MemorySpace.ANY),
        out_shape=jax.ShapeDtypeStruct((n, out_dim), jnp.float32),
        scratch_shapes=[
            pltpu.VMEM((3, CH_MAX, hidden), jnp.float32),
            pltpu.VMEM((2, CH_MAX, out_dim), jnp.float32),
            pltpu.SemaphoreType.DMA((3, 2)),
            pltpu.SemaphoreType.DMA((2, 2)),
        ],
    )(x, W, b2)
